# baseline (speedup 1.0000x reference)
import jax
import jax.numpy as jnp
from jax.experimental import pallas as pl


def _shift_kernel(x_ref, o_ref):
    o_ref[...] = jnp.where(x_ref[...] > 0, 1.0, -1.0).astype(jnp.float32)


def kernel(inputs):
    m, n = inputs.shape
    bm = 2048
    grid = (m // bm,)
    return pl.pallas_call(
        _shift_kernel,
        grid=grid,
        in_specs=[pl.BlockSpec((bm, n), lambda i: (i, 0))],
        out_specs=pl.BlockSpec((bm, n), lambda i: (i, 0)),
        out_shape=jax.ShapeDtypeStruct((m, n), jnp.float32),
    )(inputs)


# speedup vs baseline: 1.0199x; 1.0199x over previous
import jax
import jax.numpy as jnp
from jax.experimental import pallas as pl


def _shift_kernel(x_ref, o_ref):
    o_ref[...] = jnp.where(x_ref[...] > 0, 1.0, -1.0).astype(jnp.float32)


def kernel(inputs):
    m, n = inputs.shape
    bm = 4096
    grid = (m // bm,)
    return pl.pallas_call(
        _shift_kernel,
        grid=grid,
        in_specs=[pl.BlockSpec((bm, n), lambda i: (i, 0))],
        out_specs=pl.BlockSpec((bm, n), lambda i: (i, 0)),
        out_shape=jax.ShapeDtypeStruct((m, n), jnp.float32),
    )(inputs)
